# Initial kernel scaffold; baseline (speedup 1.0000x reference)
import jax
import jax.numpy as jnp
from jax.experimental import pallas as pl


def kernel(x, full_index, full_conn, sqrt_deg, qkv_w, qkv_b, conn_w, conn_b, Aw, Bw, deg_coef, ffn1_w, ffn1_b, ffn2_w, ffn2_b, bn1_g, bn1_b, bn2_g, bn2_b):
    raise NotImplementedError("write your pallas kernel here")


# speedup vs baseline: 51.2247x; 51.2247x over previous
import functools
import jax
import jax.numpy as jnp
import numpy as np
from jax import lax
from jax.experimental import pallas as pl
from jax.experimental.pallas import tpu as pltpu
from jax.experimental.pallas import tpu_sc as plsc

N = 10000
E = 320000
D = 128
H = 8
DH = 16
CLAMP = 5.0

NC = 2
NS = 16
NW = NC * NS
EPW = E // NW


def _qkv_body(x_ref, w_ref, o_ref):
    o_ref[...] = jnp.dot(x_ref[...], w_ref[...],
                         preferred_element_type=jnp.float32)


def _qkv(x, wqkv):
    return pl.pallas_call(
        _qkv_body,
        out_shape=jax.ShapeDtypeStruct((N, 3 * D), jnp.float32),
    )(x, wqkv)


NROW = E // 128
RPT = NROW // NW
GK = 6


def _gather_body(qh, kh, vh, dst2, src2, qd_o, ks_o, vs_o,
                 idx_d, idx_s, rows_v, sem):
    wid = lax.axis_index("c") * NS + lax.axis_index("s")
    start = wid * RPT
    pltpu.sync_copy(dst2.at[pl.ds(start, RPT)], idx_d)
    pltpu.sync_copy(src2.at[pl.ds(start, RPT)], idx_s)

    def chunk(i, _):
        row0 = i * GK
        for table, idxref, out in ((qh, idx_d, qd_o), (kh, idx_s, ks_o),
                                   (vh, idx_s, vs_o)):
            cps = [pltpu.async_copy(table.at[idxref.at[row0 + j]],
                                    rows_v.at[pl.ds(j * 128, 128)], sem)
                   for j in range(GK)]
            for cp in cps:
                cp.wait()
            pltpu.sync_copy(rows_v, out.at[pl.ds((start + row0) * 128,
                                                 GK * 128)])
        return 0

    lax.fori_loop(0, RPT // GK, chunk, 0)

    @pl.when(wid < NROW - NW * RPT)
    def _():
        row = NW * RPT + wid
        pltpu.sync_copy(dst2.at[pl.ds(row, 1)], idx_d.at[pl.ds(0, 1)])
        pltpu.sync_copy(src2.at[pl.ds(row, 1)], idx_s.at[pl.ds(0, 1)])
        for table, idxref, out in ((qh, idx_d, qd_o), (kh, idx_s, ks_o),
                                   (vh, idx_s, vs_o)):
            pltpu.async_copy(table.at[idxref.at[0]],
                             rows_v.at[pl.ds(0, 128)], sem).wait()
            pltpu.sync_copy(rows_v.at[pl.ds(0, 128)],
                            out.at[pl.ds(row * 128, 128)])


def _sc_gather(qh, kh, vh, dst2, src2):
    mesh = plsc.VectorSubcoreMesh(core_axis_name="c", subcore_axis_name="s")
    f = pl.kernel(
        _gather_body,
        out_type=(
            jax.ShapeDtypeStruct((E, D), jnp.float32),
            jax.ShapeDtypeStruct((E, D), jnp.float32),
            jax.ShapeDtypeStruct((E, D), jnp.float32),
        ),
        mesh=mesh,
        scratch_types=[
            pltpu.VMEM((RPT, 128), jnp.int32),
            pltpu.VMEM((RPT, 128), jnp.int32),
            pltpu.VMEM((GK * 128, D), jnp.float32),
            pltpu.SemaphoreType.DMA,
        ],
        compiler_params=pltpu.CompilerParams(use_tc_tiling_on_sc=False),
    )
    return f(qh, kh, vh, dst2, src2)


EB = 512


def _edge_body(fc_ref, qk_a_ref, qk_b_ref, v_ref, wc_ref, bc_ref, w2_ref,
               a2_ref, s_ref, conn2_ref, w_ref, contrib_ref):
    fc = fc_ref[...]
    eh = jnp.dot(fc, wc_ref[...], preferred_element_type=jnp.float32)
    eh = eh + bc_ref[...]
    ew = eh[:, :D]
    ebias = eh[:, D:]
    conn = (qk_a_ref[...] + qk_b_ref[...]) * ew
    conn = jnp.sign(conn) * jnp.sqrt(jnp.abs(conn))
    conn = jnp.maximum(conn + ebias, 0.0)
    conn2_ref[...] = jnp.dot(conn, w2_ref[...],
                             preferred_element_type=jnp.float32)
    conn2 = jnp.maximum(conn2_ref[...] + fc, 0.0)
    conn2_ref[...] = conn2
    score = jnp.dot(conn, a2_ref[...], preferred_element_type=jnp.float32)
    w = jnp.exp(jnp.clip(score, -CLAMP, CLAMP))
    w_ref[...] = w
    wide = jnp.dot(w, s_ref[...], preferred_element_type=jnp.float32)
    contrib_ref[...] = wide * (v_ref[...] + conn2)


def _tc_edge(fc, qd, ks, vs, wc, bc, w2, a2, sel):
    grid = (E // EB,)
    eb_spec = pl.BlockSpec((EB, D), lambda i: (i, 0))
    w_spec = pl.BlockSpec((EB, H), lambda i: (i, 0))
    full = lambda shape: pl.BlockSpec(shape, lambda i: tuple(0 for _ in shape))
    return pl.pallas_call(
        _edge_body,
        grid=grid,
        in_specs=[eb_spec, eb_spec, eb_spec, eb_spec,
                  full((D, 2 * D)), full((1, 2 * D)), full((D, D)),
                  full((D, H)), full((H, D))],
        out_specs=[eb_spec, w_spec, eb_spec],
        out_shape=(
            jax.ShapeDtypeStruct((E, D), jnp.float32),
            jax.ShapeDtypeStruct((E, H), jnp.float32),
            jax.ShapeDtypeStruct((E, D), jnp.float32),
        ),
        compiler_params=pltpu.CompilerParams(
            dimension_semantics=("arbitrary",),
        ),
    )(fc, qd, ks, vs, wc, bc, w2, a2, sel)


SK = 2


def _scatter_body(contrib, w, dst2, zacc, zs, acc_o, s_o, acc_sh, s_sh,
                  idx_d, rows_v, w_v):
    c = lax.axis_index("c")
    s = lax.axis_index("s")
    wid = c * NS + s
    start = wid * RPT

    @pl.when(s == 0)
    def _():
        pltpu.sync_copy(zacc, acc_sh)
        pltpu.sync_copy(zs, s_sh)
    plsc.subcore_barrier()

    pltpu.sync_copy(dst2.at[pl.ds(start, RPT)], idx_d)

    def chunk(i, _):
        row0 = i * SK
        pltpu.sync_copy(contrib.at[pl.ds((start + row0) * 128, SK * 128)],
                        rows_v)
        pltpu.sync_copy(w.at[pl.ds((start + row0) * 128, SK * 128)], w_v)
        for j in range(SK):
            pltpu.sync_copy(rows_v.at[pl.ds(j * 128, 128)],
                            acc_sh.at[idx_d.at[row0 + j]], add=True)
            pltpu.sync_copy(w_v.at[pl.ds(j * 128, 128)],
                            s_sh.at[idx_d.at[row0 + j]], add=True)
        return 0

    lax.fori_loop(0, RPT // SK, chunk, 0)

    @pl.when(wid < NROW - NW * RPT)
    def _():
        row = NW * RPT + wid
        pltpu.sync_copy(dst2.at[pl.ds(row, 1)], idx_d.at[pl.ds(0, 1)])
        pltpu.sync_copy(contrib.at[pl.ds(row * 128, 128)],
                        rows_v.at[pl.ds(0, 128)])
        pltpu.sync_copy(w.at[pl.ds(row * 128, 128)], w_v.at[pl.ds(0, 128)])
        pltpu.sync_copy(rows_v.at[pl.ds(0, 128)], acc_sh.at[idx_d.at[0]],
                        add=True)
        pltpu.sync_copy(w_v.at[pl.ds(0, 128)], s_sh.at[idx_d.at[0]],
                        add=True)

    plsc.subcore_barrier()

    @pl.when(s == 0)
    def _():
        pltpu.sync_copy(acc_sh, acc_o.at[c])
        pltpu.sync_copy(s_sh, s_o.at[c])


def _sc_scatter(contrib, w, dst2, zacc, zs):
    mesh = plsc.VectorSubcoreMesh(core_axis_name="c", subcore_axis_name="s")
    f = pl.kernel(
        _scatter_body,
        out_type=(
            jax.ShapeDtypeStruct((NC, N, D), jnp.float32),
            jax.ShapeDtypeStruct((NC, N, H), jnp.float32),
        ),
        mesh=mesh,
        scratch_types=[
            pltpu.VMEM_SHARED((N, D), jnp.float32),
            pltpu.VMEM_SHARED((N, H), jnp.float32),
            pltpu.VMEM((RPT, 128), jnp.int32),
            pltpu.VMEM((SK * 128, D), jnp.float32),
            pltpu.VMEM((SK * 128, H), jnp.float32),
        ],
        compiler_params=pltpu.CompilerParams(use_tc_tiling_on_sc=False),
    )
    return f(contrib, w, dst2, zacc, zs)


def _node_body(acc_ref, s_ref, x_ref, sd_ref, sel_ref, c0_ref, c1_ref,
               f1w_ref, f1b_ref, f2w_ref, f2b_ref, g1_ref, b1_ref,
               g2_ref, b2_ref, out_ref):
    acc = acc_ref[0] + acc_ref[1]
    sden = s_ref[0] + s_ref[1]
    swide = jnp.dot(sden, sel_ref[...], preferred_element_type=jnp.float32)
    nh = acc / (swide + 1e-16)
    x = x_ref[...]
    nh = nh * (c0_ref[...] + sd_ref[...] * c1_ref[...])
    nh = nh + x
    h_res = nh

    m = jnp.mean(nh, axis=0, keepdims=True)
    v = jnp.mean((nh - m) * (nh - m), axis=0, keepdims=True)
    nh = (nh - m) / jnp.sqrt(v + 1e-5) * g1_ref[...] + b1_ref[...]

    nh = jnp.dot(nh, f1w_ref[...], preferred_element_type=jnp.float32)
    nh = jnp.maximum(nh + f1b_ref[...], 0.0)
    nh = jnp.dot(nh, f2w_ref[...], preferred_element_type=jnp.float32)
    nh = nh + f2b_ref[...] + h_res

    m2 = jnp.mean(nh, axis=0, keepdims=True)
    v2 = jnp.mean((nh - m2) * (nh - m2), axis=0, keepdims=True)
    out_ref[...] = (nh - m2) / jnp.sqrt(v2 + 1e-5) * g2_ref[...] + b2_ref[...]


def _tc_node(acc, s, x, sqrt_deg, sel, c0, c1, f1w, f1b, f2w, f2b,
             g1, b1, g2, b2):
    return pl.pallas_call(
        _node_body,
        out_shape=jax.ShapeDtypeStruct((N, D), jnp.float32),
        compiler_params=pltpu.CompilerParams(
            vmem_limit_bytes=120 * 1024 * 1024,
        ),
    )(acc, s, x, sqrt_deg, sel, c0, c1, f1w, f1b, f2w, f2b, g1, b1, g2, b2)


def kernel(x, full_index, full_conn, sqrt_deg, qkv_w, qkv_b, conn_w, conn_b,
           Aw, Bw, deg_coef, ffn1_w, ffn1_b, ffn2_w, ffn2_b, bn1_g, bn1_b,
           bn2_g, bn2_b):
    wqkv = qkv_w.T
    wc = conn_w.T
    bc = conn_b.reshape(1, 2 * D)
    hh = jnp.arange(D) // DH
    blk = (hh[:, None] == hh[None, :]).astype(jnp.float32)
    w2 = blk * jnp.tile(Bw.transpose(1, 0, 2).reshape(H * DH, DH), (1, H))
    a2 = (jnp.arange(H)[None, :] == hh[:, None]).astype(jnp.float32) * \
        Aw[:, :, 0].T.reshape(D, 1)
    sel = (hh[None, :] == jnp.arange(H)[:, None]).astype(jnp.float32)
    c0 = deg_coef[0, :, 0].reshape(1, D)
    c1 = deg_coef[0, :, 1].reshape(1, D)

    dst2 = full_index[0].reshape(NROW, 128)
    src2 = full_index[1].reshape(NROW, 128)
    zacc = jnp.zeros((N, D), jnp.float32)
    zs = jnp.zeros((N, H), jnp.float32)

    qkv = _qkv(x, wqkv)
    qh = qkv[:, :D]
    kh = qkv[:, D:2 * D]
    vh = qkv[:, 2 * D:]

    qd, ks, vs = _sc_gather(qh, kh, vh, dst2, src2)
    conn2, w, contrib = _tc_edge(full_conn, qd, ks, vs, wc, bc, w2, a2, sel)
    acc, s = _sc_scatter(contrib, w, dst2, zacc, zs)
    nh = _tc_node(acc, s, x, sqrt_deg, sel, c0, c1,
                  ffn1_w.T, ffn1_b.reshape(1, 2 * D),
                  ffn2_w.T, ffn2_b.reshape(1, D),
                  bn1_g.reshape(1, D), bn1_b.reshape(1, D),
                  bn2_g.reshape(1, D), bn2_b.reshape(1, D))
    return nh, conn2
